# baseline (speedup 1.0000x reference)
import functools

import jax
import jax.numpy as jnp
from jax import lax
from jax.experimental import pallas as pl
from jax.experimental.pallas import tpu as pltpu
from jax.experimental.pallas import tpu_sc as plsc

_B = 16
_H = 512
_W = 512
_S = 4096
_NW = 32
_SPW = (_B * _S) // _NW
_LANES = 16
_CHUNKS = _SPW // _LANES
_GCH = 256
_NGATHER = _SPW // _GCH
_ROWS_PER_IMG = (_H * _W) // _LANES


def _round_half_even_idx(x_f32, limit):
    del limit
    t = x_f32 + jnp.float32(0.5)
    f = t.astype(jnp.int32)
    tie_odd = (f.astype(jnp.float32) == t) & ((f & 1) == 1)
    return f - jnp.where(tie_odd, jnp.int32(1), jnp.int32(0))


def _sc_body(pred_hbm, tgt_hbm, u_hbm, v_hbm, out_hbm,
             u_v, v_v, row_v, lane_v, prow_v, trow_v, out_v, sem):
    wid = lax.axis_index("s") * 2 + lax.axis_index("c")
    b = wid // 2
    base = wid * _SPW

    cu = pltpu.async_copy(u_hbm.at[pl.ds(base, _SPW)], u_v, sem)
    cv = pltpu.async_copy(v_hbm.at[pl.ds(base, _SPW)], v_v, sem)
    cu.wait()
    cv.wait()

    row_base = b * _ROWS_PER_IMG
    _PER_BLK = _GCH // _LANES

    def idx_block(j, _):
        for k in range(_PER_BLK):
            off = j * _GCH + k * _LANES
            u16 = u_v[pl.ds(off, _LANES)]
            v16 = v_v[pl.ds(off, _LANES)]
            x = _round_half_even_idx(u16 * jnp.float32(_W - 1), _W - 1)
            y = _round_half_even_idx(v16 * jnp.float32(_H - 1), _H - 1)
            p = ((y >> 3) * 4 + (x >> 7)) * 1024 + (y & 7) * 128 + (x & 127)
            row_v[pl.ds(off, _LANES)] = row_base + (p >> 4)
            lane_v[pl.ds(off, _LANES)] = p & 15
        sl = pl.ds(j * _GCH, _GCH)
        pltpu.async_copy(pred_hbm.at[row_v.at[sl]], prow_v.at[sl], sem)
        pltpu.async_copy(tgt_hbm.at[row_v.at[sl]], trow_v.at[sl], sem)
        return _

    lax.fori_loop(0, _NGATHER, idx_block, None)

    iota16 = lax.iota(jnp.int32, _LANES)

    def acc_block(j, acc):
        sl = pl.ds(j * _GCH, _GCH)
        pltpu.make_async_copy(pred_hbm.at[row_v.at[sl]], prow_v.at[sl],
                              sem).wait()
        pltpu.make_async_copy(tgt_hbm.at[row_v.at[sl]], trow_v.at[sl],
                              sem).wait()
        base0 = iota16 + j * _GCH
        for k in range(_PER_BLK):
            off = j * _GCH + k * _LANES
            idx0 = base0 + k * _LANES
            lanes = lane_v[pl.ds(off, _LANES)]
            pv = plsc.load_gather(prow_v, [idx0, lanes])
            tv = plsc.load_gather(trow_v, [idx0, lanes])
            d = pv - tv
            acc = acc + d * d
        return acc

    acc = lax.fori_loop(0, _NGATHER, acc_block,
                        jnp.zeros((_LANES,), jnp.float32))
    total = jnp.sum(acc, axis=0)
    out_v[...] = jnp.full((_LANES,), total, jnp.float32)
    pltpu.sync_copy(out_v, out_hbm.at[wid])


_sc_gather_mse = functools.partial(
    pl.kernel,
    mesh=plsc.VectorSubcoreMesh(core_axis_name="c", subcore_axis_name="s"),
    out_type=jax.ShapeDtypeStruct((_NW, _LANES), jnp.float32),
    scratch_types=[
        pltpu.VMEM((_SPW,), jnp.float32),
        pltpu.VMEM((_SPW,), jnp.float32),
        pltpu.VMEM((_SPW,), jnp.int32),
        pltpu.VMEM((_SPW,), jnp.int32),
        pltpu.VMEM((_SPW, _LANES), jnp.float32),
        pltpu.VMEM((_SPW, _LANES), jnp.float32),
        pltpu.VMEM((_LANES,), jnp.float32),
        pltpu.SemaphoreType.DMA,
    ],
    compiler_params=pltpu.CompilerParams(needs_layout_passes=False,
                                         use_tc_tiling_on_sc=False),
)(_sc_body)


def _finish_body(p_ref, o_ref):
    o_ref[0, 0] = jnp.sum(p_ref[...]) * jnp.float32(1.0 / (_B * _S * _LANES))


_finish = pl.pallas_call(
    _finish_body,
    out_shape=jax.ShapeDtypeStruct((1, 1), jnp.float32),
    out_specs=pl.BlockSpec(memory_space=pltpu.SMEM),
)


def _physical_view(a):
    return (a.reshape(_B, _H // 8, 8, _W // 128, 128)
             .transpose(0, 1, 3, 2, 4)
             .reshape(_B * _ROWS_PER_IMG, _LANES))


def kernel(pred, target, sampled_coords):
    pred2d = _physical_view(pred)
    tgt2d = _physical_view(target)
    u = sampled_coords[:, :, 0].reshape(_B * _S)
    v = sampled_coords[:, :, 1].reshape(_B * _S)
    partials = _sc_gather_mse(pred2d, tgt2d, u, v)
    return _finish(partials.reshape(_NW * _LANES))[0, 0]


# speedup vs baseline: 1.0187x; 1.0187x over previous
import functools

import jax
import jax.numpy as jnp
from jax import lax
from jax.experimental import pallas as pl
from jax.experimental.pallas import tpu as pltpu
from jax.experimental.pallas import tpu_sc as plsc

_B = 16
_H = 512
_W = 512
_S = 4096
_NW = 32
_SPW = (_B * _S) // _NW
_LANES = 16
_CHUNKS = _SPW // _LANES
_GCH = 128
_NGATHER = _SPW // _GCH
_ROWS_PER_IMG = (_H * _W) // _LANES


def _round_half_even_idx(x_f32, limit):
    del limit
    t = x_f32 + jnp.float32(0.5)
    f = t.astype(jnp.int32)
    tie_odd = (f.astype(jnp.float32) == t) & ((f & 1) == 1)
    return f - jnp.where(tie_odd, jnp.int32(1), jnp.int32(0))


def _sc_body(pred_hbm, tgt_hbm, u_hbm, v_hbm, out_hbm,
             u_v, v_v, row_v, lane_v, prow_v, trow_v, out_v, sem):
    wid = lax.axis_index("s") * 2 + lax.axis_index("c")
    b = wid // 2
    base = wid * _SPW

    cu = pltpu.async_copy(u_hbm.at[pl.ds(base, _SPW)], u_v, sem)
    cv = pltpu.async_copy(v_hbm.at[pl.ds(base, _SPW)], v_v, sem)
    cu.wait()
    cv.wait()

    row_base = b * _ROWS_PER_IMG
    _PER_BLK = _GCH // _LANES

    def idx_block(j, _):
        for k in range(_PER_BLK):
            off = j * _GCH + k * _LANES
            u16 = u_v[pl.ds(off, _LANES)]
            v16 = v_v[pl.ds(off, _LANES)]
            x = _round_half_even_idx(u16 * jnp.float32(_W - 1), _W - 1)
            y = _round_half_even_idx(v16 * jnp.float32(_H - 1), _H - 1)
            p = ((y >> 3) * 4 + (x >> 7)) * 1024 + (y & 7) * 128 + (x & 127)
            row_v[pl.ds(off, _LANES)] = row_base + (p >> 4)
            lane_v[pl.ds(off, _LANES)] = p & 15
        sl = pl.ds(j * _GCH, _GCH)
        pltpu.async_copy(pred_hbm.at[row_v.at[sl]], prow_v.at[sl], sem)
        pltpu.async_copy(tgt_hbm.at[row_v.at[sl]], trow_v.at[sl], sem)
        return _

    plsc.parallel_loop(0, _NGATHER)(lambda j: idx_block(j, None))

    iota16 = lax.iota(jnp.int32, _LANES)

    def acc_block(j, acc):
        sl = pl.ds(j * _GCH, _GCH)
        pltpu.make_async_copy(pred_hbm.at[row_v.at[sl]], prow_v.at[sl],
                              sem).wait()
        pltpu.make_async_copy(tgt_hbm.at[row_v.at[sl]], trow_v.at[sl],
                              sem).wait()
        base0 = iota16 + j * _GCH
        for k in range(_PER_BLK):
            off = j * _GCH + k * _LANES
            idx0 = base0 + k * _LANES
            lanes = lane_v[pl.ds(off, _LANES)]
            pv = plsc.load_gather(prow_v, [idx0, lanes])
            tv = plsc.load_gather(trow_v, [idx0, lanes])
            d = pv - tv
            acc = acc + d * d
        return acc

    acc = lax.fori_loop(0, _NGATHER, acc_block,
                        jnp.zeros((_LANES,), jnp.float32))
    total = jnp.sum(acc, axis=0)
    out_v[...] = jnp.full((_LANES,), total, jnp.float32)
    pltpu.sync_copy(out_v, out_hbm.at[wid])


_sc_gather_mse = functools.partial(
    pl.kernel,
    mesh=plsc.VectorSubcoreMesh(core_axis_name="c", subcore_axis_name="s"),
    out_type=jax.ShapeDtypeStruct((_NW, _LANES), jnp.float32),
    scratch_types=[
        pltpu.VMEM((_SPW,), jnp.float32),
        pltpu.VMEM((_SPW,), jnp.float32),
        pltpu.VMEM((_SPW,), jnp.int32),
        pltpu.VMEM((_SPW,), jnp.int32),
        pltpu.VMEM((_SPW, _LANES), jnp.float32),
        pltpu.VMEM((_SPW, _LANES), jnp.float32),
        pltpu.VMEM((_LANES,), jnp.float32),
        pltpu.SemaphoreType.DMA,
    ],
    compiler_params=pltpu.CompilerParams(needs_layout_passes=False,
                                         use_tc_tiling_on_sc=False),
)(_sc_body)


def _finish_body(p_ref, o_ref):
    o_ref[0, 0] = jnp.sum(p_ref[...]) * jnp.float32(1.0 / (_B * _S * _LANES))


_finish = pl.pallas_call(
    _finish_body,
    out_shape=jax.ShapeDtypeStruct((1, 1), jnp.float32),
    out_specs=pl.BlockSpec(memory_space=pltpu.SMEM),
)


def _physical_view(a):
    return (a.reshape(_B, _H // 8, 8, _W // 128, 128)
             .transpose(0, 1, 3, 2, 4)
             .reshape(_B * _ROWS_PER_IMG, _LANES))


def kernel(pred, target, sampled_coords):
    pred2d = _physical_view(pred)
    tgt2d = _physical_view(target)
    u = sampled_coords[:, :, 0].reshape(_B * _S)
    v = sampled_coords[:, :, 1].reshape(_B * _S)
    partials = _sc_gather_mse(pred2d, tgt2d, u, v)
    return _finish(partials.reshape(_NW * _LANES))[0, 0]
